# Initial kernel scaffold; baseline (speedup 1.0000x reference)
import functools

import jax
import jax.numpy as jnp
from jax import lax
from jax.experimental import pallas as pl
from jax.experimental.pallas import tpu as pltpu
from jax.experimental.pallas import tpu_sc as plsc

N = 10000
E = 320000
FIN = 128
HID = 128
C = 40
CP = 48

NC = 2
NS = 16
NW = NC * NS
EW = E // NW
K = 125
NJ = EW // K
RPT = N // NS
DEGW = 16

RB = 1000


def _mesh():
    return plsc.VectorSubcoreMesh(core_axis_name="c", subcore_axis_name="s")


@functools.partial(
    pl.kernel,
    out_type=jax.ShapeDtypeStruct((NC, N, DEGW), jnp.float32),
    mesh=_mesh(),
    scratch_types=[
        pltpu.VMEM((NJ, K), jnp.int32),
        pltpu.VMEM((K, DEGW), jnp.float32),
        pltpu.VMEM((RPT, DEGW), jnp.float32),
        pltpu.VMEM_SHARED((N, DEGW), jnp.float32),
    ],
)
def _deg(dstb_hbm, ones_hbm, zeros_hbm, out_hbm, dst_v, ones_v, zbuf, acc):
    c = lax.axis_index("c")
    s = lax.axis_index("s")
    w = c * NS + s
    pltpu.sync_copy(zeros_hbm, zbuf)
    pltpu.sync_copy(zbuf, acc.at[pl.ds(s * RPT, RPT)])
    pltpu.sync_copy(ones_hbm, ones_v)
    pltpu.sync_copy(dstb_hbm.at[w], dst_v)
    plsc.subcore_barrier()

    def body(j, carry):
        pltpu.sync_copy(ones_v, acc.at[dst_v.at[j]], add=True)
        return carry

    lax.fori_loop(0, NJ, body, 0)
    plsc.subcore_barrier()
    pltpu.sync_copy(acc.at[pl.ds(s * RPT, RPT)], out_hbm.at[c, pl.ds(s * RPT, RPT)])


def _make_agg(width):
    @functools.partial(
        pl.kernel,
        out_type=jax.ShapeDtypeStruct((NC, N, width), jnp.float32),
        mesh=_mesh(),
        scratch_types=[
            pltpu.VMEM((NJ, K), jnp.int32),
            pltpu.VMEM((NJ, K), jnp.int32),
            pltpu.VMEM((2, K, width), jnp.float32),
            pltpu.VMEM((K, width), jnp.float32),
            pltpu.VMEM_SHARED((N, width), jnp.float32),
            pltpu.SemaphoreType.DMA,
            pltpu.SemaphoreType.DMA,
        ],
    )
    def agg(g_hbm, srcb_hbm, dstb_hbm, zeros_hbm, out_hbm,
            src_v, dst_v, rows, zbuf, acc, sem0, sem1):
        c = lax.axis_index("c")
        s = lax.axis_index("s")
        w = c * NS + s
        pltpu.sync_copy(zeros_hbm, zbuf)
        for r in range(RPT // K):
            pltpu.sync_copy(zbuf, acc.at[pl.ds(s * RPT + r * K, K)])
        pltpu.sync_copy(srcb_hbm.at[w], src_v)
        pltpu.sync_copy(dstb_hbm.at[w], dst_v)
        plsc.subcore_barrier()

        pltpu.async_copy(g_hbm.at[src_v.at[0]], rows.at[0], sem0)

        def body(j2, carry):
            j = j2 * 2
            pltpu.async_copy(g_hbm.at[src_v.at[j + 1]], rows.at[1], sem1)
            pltpu.make_async_copy(g_hbm.at[src_v.at[j]], rows.at[0], sem0).wait()
            pltpu.sync_copy(rows.at[0], acc.at[dst_v.at[j]], add=True)

            @pl.when(j2 < NJ // 2 - 1)
            def _():
                pltpu.async_copy(g_hbm.at[src_v.at[j + 2]], rows.at[0], sem0)

            pltpu.make_async_copy(g_hbm.at[src_v.at[j + 1]], rows.at[1], sem1).wait()
            pltpu.sync_copy(rows.at[1], acc.at[dst_v.at[j + 1]], add=True)
            return carry

        lax.fori_loop(0, NJ // 2, body, 0)
        plsc.subcore_barrier()
        pltpu.sync_copy(acc.at[pl.ds(s * RPT, RPT)],
                        out_hbm.at[c, pl.ds(s * RPT, RPT)])

    return agg


_agg128 = _make_agg(HID)
_agg48 = _make_agg(CP)


def _dinv_of(dg_ref):
    d = dg_ref[0, :, 0:1] + dg_ref[1, :, 0:1] + 1.0
    return lax.rsqrt(jnp.maximum(d, 1.0))


def _mm1_body(x_ref, w_ref, dg_ref, o_ref):
    dinv = _dinv_of(dg_ref)
    o_ref[...] = jnp.dot(x_ref[...], w_ref[...],
                         preferred_element_type=jnp.float32) * dinv


def _mid_body(a_ref, g1_ref, dg_ref, b1_ref, w2_ref, o_ref):
    dinv = _dinv_of(dg_ref)
    z = (a_ref[0] + a_ref[1] + g1_ref[...]) * dinv + b1_ref[...]
    h = jnp.maximum(z, 0.0)
    o_ref[...] = jnp.dot(h, w2_ref[...],
                         preferred_element_type=jnp.float32) * dinv


def _fin_body(b_ref, g2_ref, dg_ref, b2_ref, o_ref):
    dinv = _dinv_of(dg_ref)
    z = (b_ref[0] + b_ref[1] + g2_ref[...]) * dinv + b2_ref[...]
    col = lax.broadcasted_iota(jnp.int32, (RB, CP), 1)
    valid = col < C
    zm = jnp.where(valid, z, -jnp.inf)
    m = jnp.max(zm, axis=1, keepdims=True)
    e = jnp.where(valid, jnp.exp(z - m), 0.0)
    ssum = jnp.sum(e, axis=1, keepdims=True)
    o_ref[...] = (z - m - jnp.log(ssum))[:, :C]


def _row_spec(width):
    return pl.BlockSpec((RB, width), lambda i: (i, 0))


def _pair_spec(width):
    return pl.BlockSpec((NC, RB, width), lambda i: (0, i, 0))


def _const_spec(shape):
    return pl.BlockSpec(shape, lambda i: (0,) * len(shape))


@jax.jit
def kernel(x, edge_index, W1, b1, W2, b2):
    srcb = edge_index[0].reshape(NW, NJ, K)
    dstb = edge_index[1].reshape(NW, NJ, K)
    ones16 = jnp.ones((K, DEGW), jnp.float32)
    zeros16 = jnp.zeros((RPT, DEGW), jnp.float32)
    zeros128 = jnp.zeros((K, HID), jnp.float32)
    zeros48 = jnp.zeros((K, CP), jnp.float32)
    W2p = jnp.pad(W2, ((0, 0), (0, CP - C)))
    b1r = b1.reshape(1, HID)
    b2r = jnp.pad(b2, (0, CP - C)).reshape(1, CP)

    dga = _deg(dstb, ones16, zeros16)

    g1 = pl.pallas_call(
        _mm1_body,
        grid=(N // RB,),
        in_specs=[_row_spec(FIN), _const_spec((FIN, HID)), _pair_spec(DEGW)],
        out_specs=_row_spec(HID),
        out_shape=jax.ShapeDtypeStruct((N, HID), jnp.float32),
    )(x, W1, dga)

    s1 = _agg128(g1, srcb, dstb, zeros128)

    g2 = pl.pallas_call(
        _mid_body,
        grid=(N // RB,),
        in_specs=[_pair_spec(HID), _row_spec(HID), _pair_spec(DEGW),
                  _const_spec((1, HID)), _const_spec((HID, CP))],
        out_specs=_row_spec(CP),
        out_shape=jax.ShapeDtypeStruct((N, CP), jnp.float32),
    )(s1, g1, dga, b1r, W2p)

    s2 = _agg48(g2, srcb, dstb, zeros48)

    out = pl.pallas_call(
        _fin_body,
        grid=(N // RB,),
        in_specs=[_pair_spec(CP), _row_spec(CP), _pair_spec(DEGW),
                  _const_spec((1, CP))],
        out_specs=_row_spec(C),
        out_shape=jax.ShapeDtypeStruct((N, C), jnp.float32),
    )(s2, g2, dga, b2r)

    return out


# speedup vs baseline: 31.3981x; 31.3981x over previous
import functools

import jax
import jax.numpy as jnp
from jax import lax
from jax.experimental import pallas as pl
from jax.experimental.pallas import tpu as pltpu
from jax.experimental.pallas import tpu_sc as plsc

N = 10000
E = 320000
FIN = 128
HID = 128
HH = HID // 2
C = 40
CP = 48

NC = 2
NS = 16
NW = NC * NS
K = 125
NJ1 = E // NS // K
NJ2 = E // NW // K
DEGW = 16

RB = 1000


def _mesh():
    return plsc.VectorSubcoreMesh(core_axis_name="c", subcore_axis_name="s")


@functools.partial(
    pl.kernel,
    out_type=jax.ShapeDtypeStruct((NC, N, DEGW), jnp.float32),
    mesh=_mesh(),
    compiler_params=pltpu.CompilerParams(use_tc_tiling_on_sc=False),
    scratch_types=[
        pltpu.VMEM((NJ2, K), jnp.int32),
        pltpu.VMEM((K, DEGW), jnp.float32),
        pltpu.VMEM_SHARED((N, DEGW), jnp.float32),
    ],
)
def _deg(dstb_hbm, ones_hbm, zeros_hbm, out_hbm, dst_v, ones_v, acc):
    c = lax.axis_index("c")
    s = lax.axis_index("s")
    w = c * NS + s

    @pl.when(s == 0)
    def _():
        pltpu.sync_copy(zeros_hbm, acc)

    pltpu.sync_copy(ones_hbm, ones_v)
    pltpu.sync_copy(dstb_hbm.at[w], dst_v)
    plsc.subcore_barrier()

    def body(j, carry):
        pltpu.sync_copy(ones_v, acc.at[dst_v.at[j]], add=True)
        return carry

    lax.fori_loop(0, NJ2, body, 0)
    plsc.subcore_barrier()

    @pl.when(s == 0)
    def _():
        pltpu.sync_copy(acc, out_hbm.at[c])


@functools.partial(
    pl.kernel,
    out_type=jax.ShapeDtypeStruct((NC, N, HH), jnp.float32),
    mesh=_mesh(),
    compiler_params=pltpu.CompilerParams(use_tc_tiling_on_sc=False),
    scratch_types=[
        pltpu.VMEM((NJ1, K), jnp.int32),
        pltpu.VMEM((NJ1, K), jnp.int32),
        pltpu.VMEM((2, K, HH), jnp.float32),
        pltpu.VMEM_SHARED((N, HH), jnp.float32),
        pltpu.SemaphoreType.DMA,
        pltpu.SemaphoreType.DMA,
    ],
)
def _agg_split(g_hbm, srcb_hbm, dstb_hbm, zeros_hbm, out_hbm,
               src_v, dst_v, rows, acc, sem0, sem1):
    c = lax.axis_index("c")
    s = lax.axis_index("s")

    @pl.when(s == 0)
    def _():
        pltpu.sync_copy(zeros_hbm, acc)

    pltpu.sync_copy(srcb_hbm.at[s], src_v)
    pltpu.sync_copy(dstb_hbm.at[s], dst_v)
    plsc.subcore_barrier()

    tab = g_hbm.at[c]
    pltpu.async_copy(tab.at[src_v.at[0]], rows.at[0], sem0)

    def body(j2, carry):
        j = j2 * 2
        pltpu.async_copy(tab.at[src_v.at[j + 1]], rows.at[1], sem1)
        pltpu.make_async_copy(tab.at[src_v.at[j]], rows.at[0], sem0).wait()
        pltpu.sync_copy(rows.at[0], acc.at[dst_v.at[j]], add=True)

        @pl.when(j2 < NJ1 // 2 - 1)
        def _():
            pltpu.async_copy(tab.at[src_v.at[j + 2]], rows.at[0], sem0)

        pltpu.make_async_copy(tab.at[src_v.at[j + 1]], rows.at[1], sem1).wait()
        pltpu.sync_copy(rows.at[1], acc.at[dst_v.at[j + 1]], add=True)
        return carry

    lax.fori_loop(0, NJ1 // 2, body, 0)
    plsc.subcore_barrier()

    @pl.when(s == 0)
    def _():
        pltpu.sync_copy(acc, out_hbm.at[c])


@functools.partial(
    pl.kernel,
    out_type=jax.ShapeDtypeStruct((NC, N, CP), jnp.float32),
    mesh=_mesh(),
    compiler_params=pltpu.CompilerParams(use_tc_tiling_on_sc=False),
    scratch_types=[
        pltpu.VMEM((NJ2, K), jnp.int32),
        pltpu.VMEM((NJ2, K), jnp.int32),
        pltpu.VMEM((2, K, CP), jnp.float32),
        pltpu.VMEM_SHARED((N, CP), jnp.float32),
        pltpu.SemaphoreType.DMA,
        pltpu.SemaphoreType.DMA,
    ],
)
def _agg_full(g_hbm, srcb_hbm, dstb_hbm, zeros_hbm, out_hbm,
              src_v, dst_v, rows, acc, sem0, sem1):
    c = lax.axis_index("c")
    s = lax.axis_index("s")
    w = c * NS + s

    @pl.when(s == 0)
    def _():
        pltpu.sync_copy(zeros_hbm, acc)

    pltpu.sync_copy(srcb_hbm.at[w], src_v)
    pltpu.sync_copy(dstb_hbm.at[w], dst_v)
    plsc.subcore_barrier()

    pltpu.async_copy(g_hbm.at[src_v.at[0]], rows.at[0], sem0)

    def body(j2, carry):
        j = j2 * 2
        pltpu.async_copy(g_hbm.at[src_v.at[j + 1]], rows.at[1], sem1)
        pltpu.make_async_copy(g_hbm.at[src_v.at[j]], rows.at[0], sem0).wait()
        pltpu.sync_copy(rows.at[0], acc.at[dst_v.at[j]], add=True)

        @pl.when(j2 < NJ2 // 2 - 1)
        def _():
            pltpu.async_copy(g_hbm.at[src_v.at[j + 2]], rows.at[0], sem0)

        pltpu.make_async_copy(g_hbm.at[src_v.at[j + 1]], rows.at[1], sem1).wait()
        pltpu.sync_copy(rows.at[1], acc.at[dst_v.at[j + 1]], add=True)
        return carry

    lax.fori_loop(0, NJ2 // 2, body, 0)
    plsc.subcore_barrier()

    @pl.when(s == 0)
    def _():
        pltpu.sync_copy(acc, out_hbm.at[c])


def _dinv_of(dg_ref):
    d = dg_ref[0, :, 0:1] + dg_ref[1, :, 0:1] + 1.0
    return lax.rsqrt(jnp.maximum(d, 1.0))


def _mm1_body(x_ref, w_ref, dg_ref, o_ref):
    dinv = _dinv_of(dg_ref)
    y = jnp.dot(x_ref[...], w_ref[...],
                preferred_element_type=jnp.float32) * dinv
    o_ref[0] = y[:, :HH]
    o_ref[1] = y[:, HH:]


def _mid_body(a_ref, g1_ref, dg_ref, b1_ref, w2_ref, o_ref):
    dinv = _dinv_of(dg_ref)
    s1 = jnp.concatenate([a_ref[0], a_ref[1]], axis=1)
    g1 = jnp.concatenate([g1_ref[0], g1_ref[1]], axis=1)
    z = (s1 + g1) * dinv + b1_ref[...]
    h = jnp.maximum(z, 0.0)
    o_ref[...] = jnp.dot(h, w2_ref[...],
                         preferred_element_type=jnp.float32) * dinv


def _fin_body(b_ref, g2_ref, dg_ref, b2_ref, o_ref):
    dinv = _dinv_of(dg_ref)
    z = (b_ref[0] + b_ref[1] + g2_ref[...]) * dinv + b2_ref[...]
    col = lax.broadcasted_iota(jnp.int32, (RB, CP), 1)
    valid = col < C
    zm = jnp.where(valid, z, -jnp.inf)
    m = jnp.max(zm, axis=1, keepdims=True)
    e = jnp.where(valid, jnp.exp(z - m), 0.0)
    ssum = jnp.sum(e, axis=1, keepdims=True)
    o_ref[...] = (z - m - jnp.log(ssum))[:, :C]


def _row_spec(width):
    return pl.BlockSpec((RB, width), lambda i: (i, 0))


def _pair_spec(width):
    return pl.BlockSpec((NC, RB, width), lambda i: (0, i, 0))


def _const_spec(shape):
    return pl.BlockSpec(shape, lambda i: (0,) * len(shape))


@jax.jit
def kernel(x, edge_index, W1, b1, W2, b2):
    srcb1 = edge_index[0].reshape(NS, NJ1, K)
    dstb1 = edge_index[1].reshape(NS, NJ1, K)
    srcb2 = edge_index[0].reshape(NW, NJ2, K)
    dstb2 = edge_index[1].reshape(NW, NJ2, K)
    ones16 = jnp.ones((K, DEGW), jnp.float32)
    zeros16 = jnp.zeros((N, DEGW), jnp.float32)
    zeros64 = jnp.zeros((N, HH), jnp.float32)
    zeros48 = jnp.zeros((N, CP), jnp.float32)
    W2p = jnp.pad(W2, ((0, 0), (0, CP - C)))
    b1r = b1.reshape(1, HID)
    b2r = jnp.pad(b2, (0, CP - C)).reshape(1, CP)

    dga = _deg(dstb2, ones16, zeros16)

    g1 = pl.pallas_call(
        _mm1_body,
        grid=(N // RB,),
        in_specs=[_row_spec(FIN), _const_spec((FIN, HID)), _pair_spec(DEGW)],
        out_specs=_pair_spec(HH),
        out_shape=jax.ShapeDtypeStruct((NC, N, HH), jnp.float32),
    )(x, W1, dga)

    s1 = _agg_split(g1, srcb1, dstb1, zeros64)

    g2 = pl.pallas_call(
        _mid_body,
        grid=(N // RB,),
        in_specs=[_pair_spec(HH), _pair_spec(HH), _pair_spec(DEGW),
                  _const_spec((1, HID)), _const_spec((HID, CP))],
        out_specs=_row_spec(CP),
        out_shape=jax.ShapeDtypeStruct((N, CP), jnp.float32),
    )(s1, g1, dga, b1r, W2p)

    s2 = _agg_full(g2, srcb2, dstb2, zeros48)

    out = pl.pallas_call(
        _fin_body,
        grid=(N // RB,),
        in_specs=[_pair_spec(CP), _row_spec(CP), _pair_spec(DEGW),
                  _const_spec((1, CP))],
        out_specs=_row_spec(C),
        out_shape=jax.ShapeDtypeStruct((N, C), jnp.float32),
    )(s2, g2, dga, b2r)

    return out


# speedup vs baseline: 37.9814x; 1.2097x over previous
import functools

import jax
import jax.numpy as jnp
from jax import lax
from jax.experimental import pallas as pl
from jax.experimental.pallas import tpu as pltpu
from jax.experimental.pallas import tpu_sc as plsc

N = 10000
E = 320000
FIN = 128
HID = 128
HH = HID // 2
C = 40
CP = 48

NC = 2
NS = 16
NW = NC * NS
K = 125
NJ1 = E // NS // K
NJ2 = E // NW // K
DEGW = 16

RB = 1000


def _mesh():
    return plsc.VectorSubcoreMesh(core_axis_name="c", subcore_axis_name="s")


@functools.partial(
    pl.kernel,
    out_type=jax.ShapeDtypeStruct((NC, N, DEGW), jnp.float32),
    mesh=_mesh(),
    compiler_params=pltpu.CompilerParams(use_tc_tiling_on_sc=False),
    scratch_types=[
        pltpu.VMEM((NJ2, K), jnp.int32),
        pltpu.VMEM((K, DEGW), jnp.float32),
        pltpu.VMEM_SHARED((N, DEGW), jnp.float32),
    ],
)
def _deg(eib_hbm, ones_hbm, zeros_hbm, out_hbm, dst_v, ones_v, acc):
    c = lax.axis_index("c")
    s = lax.axis_index("s")
    w = c * NS + s

    @pl.when(s == 0)
    def _():
        pltpu.sync_copy(zeros_hbm, acc)

    pltpu.sync_copy(ones_hbm, ones_v)
    pltpu.sync_copy(eib_hbm.at[1, w], dst_v)
    plsc.subcore_barrier()

    def body(j, carry):
        pltpu.sync_copy(ones_v, acc.at[dst_v.at[j]], add=True)
        return carry

    lax.fori_loop(0, NJ2, body, 0)
    plsc.subcore_barrier()

    @pl.when(s == 0)
    def _():
        pltpu.sync_copy(acc, out_hbm.at[c])


DEPTH = 4


def _pipelined_agg(tab, acc, idx_row, dst_row, rows, sems, njt):
    for t in range(DEPTH):
        pltpu.async_copy(tab.at[idx_row(t)], rows.at[t], sems[t])

    def body(jb, carry):
        for t in range(DEPTH):
            j = jb * DEPTH + t
            pltpu.make_async_copy(tab.at[idx_row(j)], rows.at[t], sems[t]).wait()
            pltpu.sync_copy(rows.at[t], acc.at[dst_row(j)], add=True)

            @pl.when(j + DEPTH < njt)
            def _():
                pltpu.async_copy(tab.at[idx_row(j + DEPTH)], rows.at[t], sems[t])
        return carry

    lax.fori_loop(0, njt // DEPTH, body, 0)


@functools.partial(
    pl.kernel,
    out_type=jax.ShapeDtypeStruct((NC, N, HH), jnp.float32),
    mesh=_mesh(),
    compiler_params=pltpu.CompilerParams(use_tc_tiling_on_sc=False),
    scratch_types=[
        pltpu.VMEM((2, NJ2, K), jnp.int32),
        pltpu.VMEM((2, NJ2, K), jnp.int32),
        pltpu.VMEM((DEPTH, K, HH), jnp.float32),
        pltpu.VMEM_SHARED((N, HH), jnp.float32),
        [pltpu.SemaphoreType.DMA] * DEPTH,
    ],
)
def _agg_split(g_hbm, eib_hbm, zeros_hbm, out_hbm,
               src_v, dst_v, rows, acc, sems):
    c = lax.axis_index("c")
    s = lax.axis_index("s")

    @pl.when(s == 0)
    def _():
        pltpu.sync_copy(zeros_hbm, acc)

    pltpu.sync_copy(eib_hbm.at[0, pl.ds(2 * s, 2)], src_v)
    pltpu.sync_copy(eib_hbm.at[1, pl.ds(2 * s, 2)], dst_v)
    plsc.subcore_barrier()

    tab = g_hbm.at[c]
    for q in range(2):
        _pipelined_agg(tab, acc,
                       lambda j: src_v.at[q, j], lambda j: dst_v.at[q, j],
                       rows, sems, NJ2)
    plsc.subcore_barrier()

    @pl.when(s == 0)
    def _():
        pltpu.sync_copy(acc, out_hbm.at[c])


@functools.partial(
    pl.kernel,
    out_type=jax.ShapeDtypeStruct((NC, N, CP), jnp.float32),
    mesh=_mesh(),
    compiler_params=pltpu.CompilerParams(use_tc_tiling_on_sc=False),
    scratch_types=[
        pltpu.VMEM((NJ2, K), jnp.int32),
        pltpu.VMEM((NJ2, K), jnp.int32),
        pltpu.VMEM((DEPTH, K, CP), jnp.float32),
        pltpu.VMEM_SHARED((N, CP), jnp.float32),
        [pltpu.SemaphoreType.DMA] * DEPTH,
    ],
)
def _agg_full(g_hbm, eib_hbm, zeros_hbm, out_hbm,
              src_v, dst_v, rows, acc, sems):
    c = lax.axis_index("c")
    s = lax.axis_index("s")
    w = c * NS + s

    @pl.when(s == 0)
    def _():
        pltpu.sync_copy(zeros_hbm, acc)

    pltpu.sync_copy(eib_hbm.at[0, w], src_v)
    pltpu.sync_copy(eib_hbm.at[1, w], dst_v)
    plsc.subcore_barrier()

    _pipelined_agg(g_hbm, acc,
                   lambda j: src_v.at[j], lambda j: dst_v.at[j],
                   rows, sems, NJ2)
    plsc.subcore_barrier()

    @pl.when(s == 0)
    def _():
        pltpu.sync_copy(acc, out_hbm.at[c])


def _dinv_of(dg_ref):
    d = dg_ref[0, :, 0:1] + dg_ref[1, :, 0:1] + 1.0
    return lax.rsqrt(jnp.maximum(d, 1.0))


def _mm1_body(x_ref, w_ref, dg_ref, o_ref):
    dinv = _dinv_of(dg_ref)
    y = jnp.dot(x_ref[...], w_ref[...],
                preferred_element_type=jnp.float32) * dinv
    o_ref[0] = y[:, :HH]
    o_ref[1] = y[:, HH:]


def _mid_body(a_ref, g1_ref, dg_ref, b1_ref, w2_ref, o_ref):
    dinv = _dinv_of(dg_ref)
    s1 = jnp.concatenate([a_ref[0], a_ref[1]], axis=1)
    g1 = jnp.concatenate([g1_ref[0], g1_ref[1]], axis=1)
    z = (s1 + g1) * dinv + b1_ref[...]
    h = jnp.maximum(z, 0.0)
    o_ref[...] = jnp.dot(h, w2_ref[...],
                         preferred_element_type=jnp.float32) * dinv


def _fin_body(b_ref, g2_ref, dg_ref, b2_ref, o_ref):
    dinv = _dinv_of(dg_ref)
    z = (b_ref[0] + b_ref[1] + g2_ref[...]) * dinv + b2_ref[...]
    col = lax.broadcasted_iota(jnp.int32, (RB, CP), 1)
    valid = col < C
    zm = jnp.where(valid, z, -jnp.inf)
    m = jnp.max(zm, axis=1, keepdims=True)
    e = jnp.where(valid, jnp.exp(z - m), 0.0)
    ssum = jnp.sum(e, axis=1, keepdims=True)
    o_ref[...] = (z - m - jnp.log(ssum))[:, :C]


def _row_spec(width):
    return pl.BlockSpec((RB, width), lambda i: (i, 0))


def _pair_spec(width):
    return pl.BlockSpec((NC, RB, width), lambda i: (0, i, 0))


def _const_spec(shape):
    return pl.BlockSpec(shape, lambda i: (0,) * len(shape))


@jax.jit
def kernel(x, edge_index, W1, b1, W2, b2):
    eib = edge_index.reshape(2, NW, NJ2, K)
    ones16 = jnp.ones((K, DEGW), jnp.float32)
    zeros16 = jnp.zeros((N, DEGW), jnp.float32)
    zeros64 = jnp.zeros((N, HH), jnp.float32)
    zeros48 = jnp.zeros((N, CP), jnp.float32)
    W2p = jnp.pad(W2, ((0, 0), (0, CP - C)))
    b1r = b1.reshape(1, HID)
    b2r = jnp.pad(b2, (0, CP - C)).reshape(1, CP)

    dga = _deg(eib, ones16, zeros16)

    g1 = pl.pallas_call(
        _mm1_body,
        grid=(N // RB,),
        in_specs=[_row_spec(FIN), _const_spec((FIN, HID)), _pair_spec(DEGW)],
        out_specs=_pair_spec(HH),
        out_shape=jax.ShapeDtypeStruct((NC, N, HH), jnp.float32),
    )(x, W1, dga)

    s1 = _agg_split(g1, eib, zeros64)

    g2 = pl.pallas_call(
        _mid_body,
        grid=(N // RB,),
        in_specs=[_pair_spec(HH), _pair_spec(HH), _pair_spec(DEGW),
                  _const_spec((1, HID)), _const_spec((HID, CP))],
        out_specs=_row_spec(CP),
        out_shape=jax.ShapeDtypeStruct((N, CP), jnp.float32),
    )(s1, g1, dga, b1r, W2p)

    s2 = _agg_full(g2, eib, zeros48)

    out = pl.pallas_call(
        _fin_body,
        grid=(N // RB,),
        in_specs=[_pair_spec(CP), _row_spec(CP), _pair_spec(DEGW),
                  _const_spec((1, CP))],
        out_specs=_row_spec(C),
        out_shape=jax.ShapeDtypeStruct((N, C), jnp.float32),
    )(s2, g2, dga, b2r)

    return out
